# baseline (speedup 1.0000x reference)
import jax
import jax.numpy as jnp
from jax.experimental import pallas as pl


def _net_kernel(x_ref, w1_ref, b1_ref, w2_ref, b2_ref, o_ref):
    wf = jnp.dot(w1_ref[...], w2_ref[...], preferred_element_type=jnp.float32)
    bf = jnp.dot(b1_ref[...], w2_ref[...], preferred_element_type=jnp.float32) + b2_ref[...]
    wff = jnp.concatenate([wf, wf], axis=1)
    bff = jnp.concatenate([bf, bf], axis=1)
    o_ref[...] = jnp.dot(x_ref[...], wff, preferred_element_type=jnp.float32) + bff


def kernel(x, edge_index, W1, b1, W2, b2):
    n, in_dim = x.shape
    hid = W1.shape[1]
    out_dim = W2.shape[1]
    b1_2d = b1.reshape(1, hid)
    b2_2d = b2.reshape(1, out_dim)
    y = pl.pallas_call(
        _net_kernel,
        out_shape=jax.ShapeDtypeStruct((n, 2 * out_dim), x.dtype),
    )(x, W1, b1_2d, W2, b2_2d)
    return 0.5 * (y[:, :out_dim] + y[:, out_dim:])


# speedup vs baseline: 1.2370x; 1.2370x over previous
import jax
import jax.numpy as jnp
from jax.experimental import pallas as pl


def _net_kernel(x_ref, w1_ref, b1_ref, w2_ref, b2_ref, o_ref):
    wf = jnp.dot(w1_ref[...], w2_ref[...], preferred_element_type=jnp.float32)
    bf = jnp.dot(b1_ref[...], w2_ref[...], preferred_element_type=jnp.float32) + b2_ref[...]
    o_ref[...] = jnp.dot(x_ref[...], wf, preferred_element_type=jnp.float32) + bf


def kernel(x, edge_index, W1, b1, W2, b2):
    n, _ = x.shape
    hid = W1.shape[1]
    out_dim = W2.shape[1]
    return pl.pallas_call(
        _net_kernel,
        out_shape=jax.ShapeDtypeStruct((n, out_dim), x.dtype),
    )(x, W1, b1.reshape(1, hid), W2, b2.reshape(1, out_dim))
